# Initial kernel scaffold; baseline (speedup 1.0000x reference)
import jax
import jax.numpy as jnp
from jax.experimental import pallas as pl


def kernel(hidden_states, weight):
    raise NotImplementedError("write your pallas kernel here")


# speedup vs baseline: 1.4909x; 1.4909x over previous
import functools

import jax
import jax.numpy as jnp
from jax.experimental import pallas as pl
from jax.experimental.pallas import tpu as pltpu

_TOKENS_PER_BLOCK = 1024


def _router_block(x_ref, w_ref, tkp_ref, tki_ref, probs_ref, aux_ref,
                  load_acc, *, num_blocks, num_tokens, num_experts):
    i = pl.program_id(0)
    t = x_ref.shape[0]

    logits = jnp.dot(x_ref[...], w_ref[...], preferred_element_type=jnp.float32)

    m = jnp.max(logits, axis=-1, keepdims=True)
    ex = jnp.exp(logits - m)
    denom = jnp.sum(ex, axis=-1, keepdims=True)
    probs = ex / denom
    probs_ref[...] = probs

    eidx = jax.lax.broadcasted_iota(jnp.int32, (t, num_experts), 1)
    i1 = jnp.min(jnp.where(logits == m, eidx, num_experts), axis=-1,
                 keepdims=True)
    p1 = jnp.max(probs, axis=-1, keepdims=True)
    sel1 = eidx == i1
    m2 = jnp.max(jnp.where(sel1, -jnp.inf, logits), axis=-1, keepdims=True)
    i2 = jnp.min(jnp.where(jnp.logical_and(logits == m2, ~sel1), eidx,
                           num_experts), axis=-1, keepdims=True)
    p2 = jnp.max(jnp.where(sel1, -1.0, probs), axis=-1, keepdims=True)

    ssum = p1 + p2
    tkp_ref[...] = jnp.concatenate([p1 / ssum, p2 / ssum], axis=-1)
    tki_ref[...] = jnp.concatenate([i1, i2], axis=-1).astype(jnp.int32)

    block_sum = jnp.sum(probs, axis=0, keepdims=True)

    @pl.when(i == 0)
    def _():
        load_acc[...] = block_sum

    @pl.when(i > 0)
    def _():
        load_acc[...] = load_acc[...] + block_sum

    @pl.when(i == num_blocks - 1)
    def _():
        load = load_acc[...] / num_tokens
        aux_ref[...] = jnp.sum(load * jnp.log(load + 1e-09), axis=-1,
                               keepdims=True)


def kernel(hidden_states, weight):
    b, s, h = hidden_states.shape
    e = weight.shape[1]
    n = b * s
    t = _TOKENS_PER_BLOCK
    num_blocks = n // t

    x = hidden_states.reshape(n, h)

    body = functools.partial(_router_block, num_blocks=num_blocks,
                             num_tokens=n, num_experts=e)

    tkp, tki, probs, aux = pl.pallas_call(
        body,
        grid=(num_blocks,),
        in_specs=[
            pl.BlockSpec((t, h), lambda i: (i, 0)),
            pl.BlockSpec((h, e), lambda i: (0, 0)),
        ],
        out_specs=[
            pl.BlockSpec((t, 2), lambda i: (i, 0)),
            pl.BlockSpec((t, 2), lambda i: (i, 0)),
            pl.BlockSpec((t, e), lambda i: (i, 0)),
            pl.BlockSpec((1, 1), lambda i: (0, 0)),
        ],
        out_shape=[
            jax.ShapeDtypeStruct((n, 2), jnp.float32),
            jax.ShapeDtypeStruct((n, 2), jnp.int32),
            jax.ShapeDtypeStruct((n, e), jnp.float32),
            jax.ShapeDtypeStruct((1, 1), jnp.float32),
        ],
        scratch_shapes=[pltpu.VMEM((1, e), jnp.float32)],
    )(x, weight)

    top_k_probs = tkp.reshape(b, s, 2)
    top_k_indices = tki.reshape(b, s, 2)
    routing_probs = probs.reshape(b, s, e)
    aux_loss = aux[0, 0]
    expert_counts = jnp.zeros((e,), dtype=jnp.int32)
    return (top_k_probs, top_k_indices, aux_loss, expert_counts, routing_probs)


# speedup vs baseline: 1.6090x; 1.0793x over previous
import functools

import jax
import jax.numpy as jnp
from jax.experimental import pallas as pl
from jax.experimental.pallas import tpu as pltpu

_TOKENS_PER_BLOCK = 1024


def _router_block(x_ref, w_ref, tkp_ref, tki_ref, probs_ref, aux_ref,
                  load_acc, *, num_blocks, num_tokens, num_experts):
    i = pl.program_id(0)
    t = x_ref.shape[0]

    logits = jnp.dot(x_ref[...], w_ref[...], preferred_element_type=jnp.float32)

    m = jnp.max(logits, axis=-1, keepdims=True)
    ex = jnp.exp(logits - m)
    denom = jnp.sum(ex, axis=-1, keepdims=True)
    recip = 1.0 / denom
    probs = ex * recip
    probs_ref[...] = probs

    eidx = jax.lax.broadcasted_iota(
        jnp.int32, (t, num_experts), 1).astype(jnp.float32)
    big = float(num_experts)
    i1 = jnp.min(jnp.where(logits == m, eidx, big), axis=-1, keepdims=True)
    sel1 = eidx == i1
    m2 = jnp.max(jnp.where(sel1, -jnp.inf, logits), axis=-1, keepdims=True)
    i2 = jnp.min(jnp.where(jnp.logical_and(logits == m2, ~sel1), eidx, big),
                 axis=-1, keepdims=True)
    p1 = recip
    p2 = jnp.exp(m2 - m) * recip

    ssum = p1 + p2
    tkp_ref[...] = jnp.concatenate([p1 / ssum, p2 / ssum], axis=-1)
    tki_ref[...] = jnp.concatenate([i1, i2], axis=-1).astype(jnp.int32)

    block_sum = jnp.sum(probs, axis=0, keepdims=True)

    @pl.when(i == 0)
    def _():
        load_acc[...] = block_sum

    @pl.when(i > 0)
    def _():
        load_acc[...] = load_acc[...] + block_sum

    @pl.when(i == num_blocks - 1)
    def _():
        load = load_acc[...] / num_tokens
        aux_ref[...] = jnp.sum(load * jnp.log(load + 1e-09), axis=-1,
                               keepdims=True)


def kernel(hidden_states, weight):
    b, s, h = hidden_states.shape
    e = weight.shape[1]
    n = b * s
    t = _TOKENS_PER_BLOCK
    num_blocks = n // t

    x = hidden_states.reshape(n, h)

    body = functools.partial(_router_block, num_blocks=num_blocks,
                             num_tokens=n, num_experts=e)

    tkp, tki, probs, aux = pl.pallas_call(
        body,
        grid=(num_blocks,),
        in_specs=[
            pl.BlockSpec((t, h), lambda i: (i, 0)),
            pl.BlockSpec((h, e), lambda i: (0, 0)),
        ],
        out_specs=[
            pl.BlockSpec((t, 2), lambda i: (i, 0)),
            pl.BlockSpec((t, 2), lambda i: (i, 0)),
            pl.BlockSpec((t, e), lambda i: (i, 0)),
            pl.BlockSpec((1, 1), lambda i: (0, 0)),
        ],
        out_shape=[
            jax.ShapeDtypeStruct((n, 2), jnp.float32),
            jax.ShapeDtypeStruct((n, 2), jnp.int32),
            jax.ShapeDtypeStruct((n, e), jnp.float32),
            jax.ShapeDtypeStruct((1, 1), jnp.float32),
        ],
        scratch_shapes=[pltpu.VMEM((1, e), jnp.float32)],
    )(x, weight)

    top_k_probs = tkp.reshape(b, s, 2)
    top_k_indices = tki.reshape(b, s, 2)
    routing_probs = probs.reshape(b, s, e)
    aux_loss = aux[0, 0]
    expert_counts = jnp.zeros((e,), dtype=jnp.int32)
    return (top_k_probs, top_k_indices, aux_loss, expert_counts, routing_probs)


# speedup vs baseline: 1.6925x; 1.0518x over previous
import functools

import jax
import jax.numpy as jnp
from jax.experimental import pallas as pl
from jax.experimental.pallas import tpu as pltpu

_TOKENS_PER_BLOCK = 2048


def _router_block(x_ref, w_ref, tkp_ref, tki_ref, probs_ref, aux_ref,
                  load_acc, *, num_blocks, num_tokens, num_experts):
    i = pl.program_id(0)
    t = x_ref.shape[0]

    logits = jnp.dot(x_ref[...], w_ref[...], preferred_element_type=jnp.float32)

    m = jnp.max(logits, axis=-1, keepdims=True)
    ex = jnp.exp(logits - m)
    denom = jnp.sum(ex, axis=-1, keepdims=True)
    recip = 1.0 / denom
    probs = ex * recip
    probs_ref[...] = probs

    eidx = jax.lax.broadcasted_iota(
        jnp.int32, (t, num_experts), 1).astype(jnp.float32)
    big = float(num_experts)
    i1 = jnp.min(jnp.where(logits == m, eidx, big), axis=-1, keepdims=True)
    sel1 = eidx == i1
    m2 = jnp.max(jnp.where(sel1, -jnp.inf, logits), axis=-1, keepdims=True)
    i2 = jnp.min(jnp.where(jnp.logical_and(logits == m2, ~sel1), eidx, big),
                 axis=-1, keepdims=True)
    p1 = recip
    p2 = jnp.exp(m2 - m) * recip

    ssum = p1 + p2
    tkp_ref[...] = jnp.concatenate([p1 / ssum, p2 / ssum], axis=-1)
    tki_ref[...] = jnp.concatenate([i1, i2], axis=-1).astype(jnp.int32)

    block_sum = jnp.sum(probs, axis=0, keepdims=True)

    @pl.when(i == 0)
    def _():
        load_acc[...] = block_sum

    @pl.when(i > 0)
    def _():
        load_acc[...] = load_acc[...] + block_sum

    @pl.when(i == num_blocks - 1)
    def _():
        load = load_acc[...] / num_tokens
        aux_ref[...] = jnp.sum(load * jnp.log(load + 1e-09), axis=-1,
                               keepdims=True)


def kernel(hidden_states, weight):
    b, s, h = hidden_states.shape
    e = weight.shape[1]
    n = b * s
    t = _TOKENS_PER_BLOCK
    num_blocks = n // t

    x = hidden_states.reshape(n, h)

    body = functools.partial(_router_block, num_blocks=num_blocks,
                             num_tokens=n, num_experts=e)

    tkp, tki, probs, aux = pl.pallas_call(
        body,
        grid=(num_blocks,),
        in_specs=[
            pl.BlockSpec((t, h), lambda i: (i, 0)),
            pl.BlockSpec((h, e), lambda i: (0, 0)),
        ],
        out_specs=[
            pl.BlockSpec((t, 2), lambda i: (i, 0)),
            pl.BlockSpec((t, 2), lambda i: (i, 0)),
            pl.BlockSpec((t, e), lambda i: (i, 0)),
            pl.BlockSpec((1, 1), lambda i: (0, 0)),
        ],
        out_shape=[
            jax.ShapeDtypeStruct((n, 2), jnp.float32),
            jax.ShapeDtypeStruct((n, 2), jnp.int32),
            jax.ShapeDtypeStruct((n, e), jnp.float32),
            jax.ShapeDtypeStruct((1, 1), jnp.float32),
        ],
        scratch_shapes=[pltpu.VMEM((1, e), jnp.float32)],
    )(x, weight)

    top_k_probs = tkp.reshape(b, s, 2)
    top_k_indices = tki.reshape(b, s, 2)
    routing_probs = probs.reshape(b, s, e)
    aux_loss = aux[0, 0]
    expert_counts = jnp.zeros((e,), dtype=jnp.int32)
    return (top_k_probs, top_k_indices, aux_loss, expert_counts, routing_probs)


# speedup vs baseline: 1.7603x; 1.0401x over previous
import functools

import jax
import jax.numpy as jnp
from jax.experimental import pallas as pl
from jax.experimental.pallas import tpu as pltpu

_TOKENS_PER_BLOCK = 4096


def _router_block(x_ref, w_ref, tkp_ref, tki_ref, probs_ref, aux_ref,
                  load_acc, *, num_blocks, num_tokens, num_experts):
    i = pl.program_id(0)
    t = x_ref.shape[0]

    logits = jnp.dot(x_ref[...], w_ref[...], preferred_element_type=jnp.float32)

    m = jnp.max(logits, axis=-1, keepdims=True)
    ex = jnp.exp(logits - m)
    denom = jnp.sum(ex, axis=-1, keepdims=True)
    recip = 1.0 / denom
    probs = ex * recip
    probs_ref[...] = probs

    eidx = jax.lax.broadcasted_iota(
        jnp.int32, (t, num_experts), 1).astype(jnp.float32)
    big = float(num_experts)
    i1 = jnp.min(jnp.where(logits == m, eidx, big), axis=-1, keepdims=True)
    sel1 = eidx == i1
    m2 = jnp.max(jnp.where(sel1, -jnp.inf, logits), axis=-1, keepdims=True)
    i2 = jnp.min(jnp.where(jnp.logical_and(logits == m2, ~sel1), eidx, big),
                 axis=-1, keepdims=True)
    p1 = recip
    p2 = jnp.exp(m2 - m) * recip

    ssum = p1 + p2
    tkp_ref[...] = jnp.concatenate([p1 / ssum, p2 / ssum], axis=-1)
    tki_ref[...] = jnp.concatenate([i1, i2], axis=-1).astype(jnp.int32)

    block_sum = jnp.sum(probs, axis=0, keepdims=True)

    @pl.when(i == 0)
    def _():
        load_acc[...] = block_sum

    @pl.when(i > 0)
    def _():
        load_acc[...] = load_acc[...] + block_sum

    @pl.when(i == num_blocks - 1)
    def _():
        load = load_acc[...] / num_tokens
        aux_ref[...] = jnp.sum(load * jnp.log(load + 1e-09), axis=-1,
                               keepdims=True)


def kernel(hidden_states, weight):
    b, s, h = hidden_states.shape
    e = weight.shape[1]
    n = b * s
    t = _TOKENS_PER_BLOCK
    num_blocks = n // t

    x = hidden_states.reshape(n, h)

    body = functools.partial(_router_block, num_blocks=num_blocks,
                             num_tokens=n, num_experts=e)

    tkp, tki, probs, aux = pl.pallas_call(
        body,
        grid=(num_blocks,),
        in_specs=[
            pl.BlockSpec((t, h), lambda i: (i, 0)),
            pl.BlockSpec((h, e), lambda i: (0, 0)),
        ],
        out_specs=[
            pl.BlockSpec((t, 2), lambda i: (i, 0)),
            pl.BlockSpec((t, 2), lambda i: (i, 0)),
            pl.BlockSpec((t, e), lambda i: (i, 0)),
            pl.BlockSpec((1, 1), lambda i: (0, 0)),
        ],
        out_shape=[
            jax.ShapeDtypeStruct((n, 2), jnp.float32),
            jax.ShapeDtypeStruct((n, 2), jnp.int32),
            jax.ShapeDtypeStruct((n, e), jnp.float32),
            jax.ShapeDtypeStruct((1, 1), jnp.float32),
        ],
        scratch_shapes=[pltpu.VMEM((1, e), jnp.float32)],
    )(x, weight)

    top_k_probs = tkp.reshape(b, s, 2)
    top_k_indices = tki.reshape(b, s, 2)
    routing_probs = probs.reshape(b, s, e)
    aux_loss = aux[0, 0]
    expert_counts = jnp.zeros((e,), dtype=jnp.int32)
    return (top_k_probs, top_k_indices, aux_loss, expert_counts, routing_probs)
